# baseline (speedup 1.0000x reference)
import functools

import jax
import jax.numpy as jnp
from jax import lax
from jax.experimental import pallas as pl
from jax.experimental.pallas import tpu as pltpu
from jax.experimental.pallas import tpu_sc as plsc

N_NODES = 10000
HID = 128
NUM_EDGES = 320000

NUM_SC = 2
NUM_SUB = 16
NUM_W = NUM_SC * NUM_SUB
EDGE_BATCH = 125
EDGES_PER_W = NUM_EDGES // NUM_W
NUM_BATCH = EDGES_PER_W // EDGE_BATCH
N_PAD = 10112
ROWS_PER_SUB = N_PAD // NUM_SUB


PIPE = 2
SG_BATCH = 40
NUM_SG = NUM_BATCH // SG_BATCH


def _scatter_add_body(h_hbm, src_hbm, dst_hbm, zeros_hbm, out_hbm,
                      src_v, dst_v, rows, agg_shared, gsem, ssem, zsem):
    cid = lax.axis_index("c")
    sid = lax.axis_index("s")
    wid = sid * NUM_SC + cid

    row0 = sid * ROWS_PER_SUB
    zero_d = pltpu.async_copy(zeros_hbm.at[pl.ds(row0, ROWS_PER_SUB)],
                              agg_shared.at[pl.ds(row0, ROWS_PER_SUB)], zsem)

    barrier_done = False
    for s in range(NUM_SG):
        b0 = s * SG_BATCH
        pltpu.sync_copy(src_hbm.at[wid, pl.ds(b0, SG_BATCH)], src_v)
        pltpu.sync_copy(dst_hbm.at[wid, pl.ds(b0, SG_BATCH)], dst_v)
        pend_g = [None] * PIPE
        pend_s = [None] * PIPE
        for i in range(SG_BATCH):
            b = i % PIPE
            if pend_s[b] is not None:
                pend_s[b].wait()
            pend_g[b] = pltpu.async_copy(h_hbm.at[src_v.at[i]], rows[b], gsem)
            if i > 0:
                if not barrier_done:
                    zero_d.wait()
                    plsc.subcore_barrier()
                    barrier_done = True
                pb = (i - 1) % PIPE
                pend_g[pb].wait()
                pend_s[pb] = pltpu.async_copy(
                    rows[pb], agg_shared.at[dst_v.at[i - 1]], ssem, add=True)
        last = (SG_BATCH - 1) % PIPE
        pend_g[last].wait()
        pend_s[last] = pltpu.async_copy(
            rows[last], agg_shared.at[dst_v.at[SG_BATCH - 1]], ssem, add=True)
        for b in range(PIPE):
            if pend_s[b] is not None:
                pend_s[b].wait()
    plsc.subcore_barrier()

    pltpu.sync_copy(agg_shared.at[pl.ds(row0, ROWS_PER_SUB)],
                    out_hbm.at[cid, pl.ds(row0, ROWS_PER_SUB)])


@functools.cache
def _scatter_add():
    return pl.kernel(
        _scatter_add_body,
        mesh=plsc.VectorSubcoreMesh(core_axis_name="c", subcore_axis_name="s"),
        out_type=jax.ShapeDtypeStruct((NUM_SC, N_PAD, HID), jnp.float32),
        scratch_types=[
            pltpu.VMEM((SG_BATCH, EDGE_BATCH), jnp.int32),
            pltpu.VMEM((SG_BATCH, EDGE_BATCH), jnp.int32),
            [pltpu.VMEM((EDGE_BATCH, HID), jnp.float32) for _ in range(PIPE)],
            pltpu.VMEM_SHARED((N_PAD, HID), jnp.float32),
            pltpu.SemaphoreType.DMA,
            pltpu.SemaphoreType.DMA,
            pltpu.SemaphoreType.DMA,
        ],
    )


ROW_BLK = 1264
GRID = N_PAD // ROW_BLK


def _gru_block(p0, p1, h, wih_t, whh_t, bih, bhh):
    agg = p0 + p1
    gi = jnp.dot(agg, wih_t, preferred_element_type=jnp.float32) + bih
    gh = jnp.dot(h, whh_t, preferred_element_type=jnp.float32) + bhh
    r = jax.nn.sigmoid(gi[:, :HID] + gh[:, :HID])
    z = jax.nn.sigmoid(gi[:, HID:2 * HID] + gh[:, HID:2 * HID])
    n = jnp.tanh(gi[:, 2 * HID:] + r * gh[:, 2 * HID:])
    return (1.0 - z) * n + z * h


def _gru_body(p_ref, h_ref, wih_ref, whh_ref, bih_ref, bhh_ref, out_ref):
    out_ref[...] = _gru_block(p_ref[0], p_ref[1], h_ref[...],
                              wih_ref[...], whh_ref[...],
                              bih_ref[...], bhh_ref[...])


def _gru_dense_max_body(p_ref, h_ref, wih_ref, whh_ref, bih_ref, bhh_ref,
                        wd_ref, bd_ref, out_ref):
    h_new = _gru_block(p_ref[0], p_ref[1], h_ref[...],
                       wih_ref[...], whh_ref[...],
                       bih_ref[...], bhh_ref[...])
    d = jnp.dot(h_new, wd_ref[...], preferred_element_type=jnp.float32) + bd_ref[...]
    rid = (pl.program_id(0) * ROW_BLK
           + jax.lax.broadcasted_iota(jnp.int32, (ROW_BLK, 1), 0))
    d = jnp.where(rid < N_NODES, d, -jnp.inf)
    blk_max = jnp.max(d, axis=0, keepdims=True)

    @pl.when(pl.program_id(0) == 0)
    def _():
        out_ref[...] = jnp.full_like(out_ref, -jnp.inf)

    out_ref[...] = jnp.maximum(out_ref[...], blk_max)


_row_spec = pl.BlockSpec((ROW_BLK, HID), lambda i: (i, 0))
_part_spec = pl.BlockSpec((NUM_SC, ROW_BLK, HID), lambda i: (0, i, 0))
_full = lambda shape: pl.BlockSpec(shape, lambda i: tuple(0 for _ in shape))

_gru_call = pl.pallas_call(
    _gru_body,
    grid=(GRID,),
    in_specs=[
        _part_spec,
        _row_spec,
        _full((HID, 3 * HID)),
        _full((HID, 3 * HID)),
        _full((1, 3 * HID)),
        _full((1, 3 * HID)),
    ],
    out_specs=_row_spec,
    out_shape=jax.ShapeDtypeStruct((N_PAD, HID), jnp.float32),
)

_gru_dense_max_call = pl.pallas_call(
    _gru_dense_max_body,
    grid=(GRID,),
    in_specs=[
        _part_spec,
        _row_spec,
        _full((HID, 3 * HID)),
        _full((HID, 3 * HID)),
        _full((1, 3 * HID)),
        _full((1, 3 * HID)),
        _full((HID, HID)),
        _full((1, HID)),
    ],
    out_specs=pl.BlockSpec((1, HID), lambda i: (0, 0)),
    out_shape=jax.ShapeDtypeStruct((1, HID), jnp.float32),
)


def kernel(x, edge_index,
           W_ih_0, W_hh_0, b_ih_0, b_hh_0,
           W_ih_1, W_hh_1, b_ih_1, b_hh_1,
           W_ih_2, W_hh_2, b_ih_2, b_hh_2,
           W_dense, b_dense):
    src = edge_index[0].astype(jnp.int32).reshape(NUM_W, NUM_BATCH, EDGE_BATCH)
    dst = edge_index[1].astype(jnp.int32).reshape(NUM_W, NUM_BATCH, EDGE_BATCH)
    zeros = jnp.zeros((N_PAD, HID), jnp.float32)
    x_pad = jnp.pad(x, ((0, N_PAD - N_NODES), (0, 0)))

    params = [(W_ih_0, W_hh_0, b_ih_0, b_hh_0),
              (W_ih_1, W_hh_1, b_ih_1, b_hh_1),
              (W_ih_2, W_hh_2, b_ih_2, b_hh_2)]

    h = x_pad
    for layer, (W_ih, W_hh, b_ih, b_hh) in enumerate(params):
        parts = _scatter_add()(h, src, dst, zeros)
        args = (parts, h, W_ih.T, W_hh.T,
                b_ih.reshape(1, -1), b_hh.reshape(1, -1))
        if layer < 2:
            h = _gru_call(*args)
        else:
            out = _gru_dense_max_call(*args, W_dense.T,
                                      b_dense.reshape(1, -1))
    return out[0]


# speedup vs baseline: 1.0158x; 1.0158x over previous
import functools

import jax
import jax.numpy as jnp
from jax import lax
from jax.experimental import pallas as pl
from jax.experimental.pallas import tpu as pltpu
from jax.experimental.pallas import tpu_sc as plsc

N_NODES = 10000
HID = 128
NUM_EDGES = 320000

NUM_SC = 2
NUM_SUB = 16
NUM_W = NUM_SC * NUM_SUB
EDGE_BATCH = 125
EDGES_PER_W = NUM_EDGES // NUM_W
NUM_BATCH = EDGES_PER_W // EDGE_BATCH
N_PAD = 10112
ROWS_PER_SUB = N_PAD // NUM_SUB


PIPE = 2
SG_BATCH = 40
NUM_SG = NUM_BATCH // SG_BATCH


def _scatter_add_body(h_hbm, src_hbm, dst_hbm, zeros_hbm, out_hbm,
                      src_v, dst_v, rows, agg_shared, gsem, ssem, zsem):
    cid = lax.axis_index("c")
    sid = lax.axis_index("s")
    wid = sid * NUM_SC + cid

    row0 = sid * ROWS_PER_SUB
    zero_d = pltpu.async_copy(zeros_hbm.at[pl.ds(row0, ROWS_PER_SUB)],
                              agg_shared.at[pl.ds(row0, ROWS_PER_SUB)], zsem)

    barrier_done = False
    for s in range(NUM_SG):
        b0 = s * SG_BATCH
        pltpu.sync_copy(src_hbm.at[wid, pl.ds(b0, SG_BATCH)], src_v)
        pltpu.sync_copy(dst_hbm.at[wid, pl.ds(b0, SG_BATCH)], dst_v)
        pend_g = [None] * PIPE
        pend_s = [None] * PIPE
        for i in range(SG_BATCH):
            b = i % PIPE
            if pend_s[b] is not None:
                pend_s[b].wait()
            pend_g[b] = pltpu.async_copy(h_hbm.at[src_v.at[i]], rows[b], gsem)
            if i > 0:
                if not barrier_done:
                    zero_d.wait()
                    plsc.subcore_barrier()
                    barrier_done = True
                pb = (i - 1) % PIPE
                pend_g[pb].wait()
                pend_s[pb] = pltpu.async_copy(
                    rows[pb], agg_shared.at[dst_v.at[i - 1]], ssem, add=True)
        last = (SG_BATCH - 1) % PIPE
        pend_g[last].wait()
        pend_s[last] = pltpu.async_copy(
            rows[last], agg_shared.at[dst_v.at[SG_BATCH - 1]], ssem, add=True)
        for b in range(PIPE):
            if pend_s[b] is not None:
                pend_s[b].wait()
    plsc.subcore_barrier()

    pltpu.sync_copy(agg_shared.at[pl.ds(row0, ROWS_PER_SUB)],
                    out_hbm.at[cid, pl.ds(row0, ROWS_PER_SUB)])


@functools.cache
def _scatter_add():
    return pl.kernel(
        _scatter_add_body,
        mesh=plsc.VectorSubcoreMesh(core_axis_name="c", subcore_axis_name="s"),
        out_type=jax.ShapeDtypeStruct((NUM_SC, N_PAD, HID), jnp.float32),
        scratch_types=[
            pltpu.VMEM((SG_BATCH, EDGE_BATCH), jnp.int32),
            pltpu.VMEM((SG_BATCH, EDGE_BATCH), jnp.int32),
            [pltpu.VMEM((EDGE_BATCH, HID), jnp.float32) for _ in range(PIPE)],
            pltpu.VMEM_SHARED((N_PAD, HID), jnp.float32),
            pltpu.SemaphoreType.DMA,
            pltpu.SemaphoreType.DMA,
            pltpu.SemaphoreType.DMA,
        ],
    )


ROW_BLK = 2528
GRID = N_PAD // ROW_BLK


def _gru_block(p0, p1, h, wih_t, whh_t, bih, bhh):
    agg = p0 + p1
    gi = jnp.dot(agg, wih_t, preferred_element_type=jnp.float32) + bih
    gh = jnp.dot(h, whh_t, preferred_element_type=jnp.float32) + bhh
    r = jax.nn.sigmoid(gi[:, :HID] + gh[:, :HID])
    z = jax.nn.sigmoid(gi[:, HID:2 * HID] + gh[:, HID:2 * HID])
    n = jnp.tanh(gi[:, 2 * HID:] + r * gh[:, 2 * HID:])
    return (1.0 - z) * n + z * h


def _gru_body(p_ref, h_ref, wih_ref, whh_ref, bih_ref, bhh_ref, out_ref):
    out_ref[...] = _gru_block(p_ref[0], p_ref[1], h_ref[...],
                              wih_ref[...], whh_ref[...],
                              bih_ref[...], bhh_ref[...])


def _gru_dense_max_body(p_ref, h_ref, wih_ref, whh_ref, bih_ref, bhh_ref,
                        wd_ref, bd_ref, out_ref):
    h_new = _gru_block(p_ref[0], p_ref[1], h_ref[...],
                       wih_ref[...], whh_ref[...],
                       bih_ref[...], bhh_ref[...])
    d = jnp.dot(h_new, wd_ref[...], preferred_element_type=jnp.float32) + bd_ref[...]
    rid = (pl.program_id(0) * ROW_BLK
           + jax.lax.broadcasted_iota(jnp.int32, (ROW_BLK, 1), 0))
    d = jnp.where(rid < N_NODES, d, -jnp.inf)
    blk_max = jnp.max(d, axis=0, keepdims=True)

    @pl.when(pl.program_id(0) == 0)
    def _():
        out_ref[...] = jnp.full_like(out_ref, -jnp.inf)

    out_ref[...] = jnp.maximum(out_ref[...], blk_max)


_row_spec = pl.BlockSpec((ROW_BLK, HID), lambda i: (i, 0))
_part_spec = pl.BlockSpec((NUM_SC, ROW_BLK, HID), lambda i: (0, i, 0))
_full = lambda shape: pl.BlockSpec(shape, lambda i: tuple(0 for _ in shape))

_gru_call = pl.pallas_call(
    _gru_body,
    grid=(GRID,),
    in_specs=[
        _part_spec,
        _row_spec,
        _full((HID, 3 * HID)),
        _full((HID, 3 * HID)),
        _full((1, 3 * HID)),
        _full((1, 3 * HID)),
    ],
    out_specs=_row_spec,
    out_shape=jax.ShapeDtypeStruct((N_PAD, HID), jnp.float32),
)

_gru_dense_max_call = pl.pallas_call(
    _gru_dense_max_body,
    grid=(GRID,),
    in_specs=[
        _part_spec,
        _row_spec,
        _full((HID, 3 * HID)),
        _full((HID, 3 * HID)),
        _full((1, 3 * HID)),
        _full((1, 3 * HID)),
        _full((HID, HID)),
        _full((1, HID)),
    ],
    out_specs=pl.BlockSpec((1, HID), lambda i: (0, 0)),
    out_shape=jax.ShapeDtypeStruct((1, HID), jnp.float32),
)


def kernel(x, edge_index,
           W_ih_0, W_hh_0, b_ih_0, b_hh_0,
           W_ih_1, W_hh_1, b_ih_1, b_hh_1,
           W_ih_2, W_hh_2, b_ih_2, b_hh_2,
           W_dense, b_dense):
    src = edge_index[0].astype(jnp.int32).reshape(NUM_W, NUM_BATCH, EDGE_BATCH)
    dst = edge_index[1].astype(jnp.int32).reshape(NUM_W, NUM_BATCH, EDGE_BATCH)
    zeros = jnp.zeros((N_PAD, HID), jnp.float32)
    x_pad = jnp.pad(x, ((0, N_PAD - N_NODES), (0, 0)))

    params = [(W_ih_0, W_hh_0, b_ih_0, b_hh_0),
              (W_ih_1, W_hh_1, b_ih_1, b_hh_1),
              (W_ih_2, W_hh_2, b_ih_2, b_hh_2)]

    h = x_pad
    for layer, (W_ih, W_hh, b_ih, b_hh) in enumerate(params):
        parts = _scatter_add()(h, src, dst, zeros)
        args = (parts, h, W_ih.T, W_hh.T,
                b_ih.reshape(1, -1), b_hh.reshape(1, -1))
        if layer < 2:
            h = _gru_call(*args)
        else:
            out = _gru_dense_max_call(*args, W_dense.T,
                                      b_dense.reshape(1, -1))
    return out[0]
